# baseline (speedup 1.0000x reference)
import jax
import jax.numpy as jnp
from jax import lax
from jax.experimental import pallas as pl
from jax.experimental.pallas import tpu as pltpu, tpu_sc as plsc

NUM_FACTORS = 16
NUM_ROWS = 1000000
BATCH = 16384
NC, NS = 2, 16
NW = NC * NS
BPW = BATCH // NW
CHUNK = 128
NCH = BPW // CHUNK


def _sc_body(users_hbm, movies_hbm, uft_hbm, mft_hbm, out_hbm,
             vidx_u, vidx_m, ucols, mcols, outv, sem_u, sem_m):
    wid = lax.axis_index("s") * NC + lax.axis_index("c")
    base = wid * BPW
    pltpu.sync_copy(users_hbm.at[pl.ds(base, BPW)], vidx_u)
    pltpu.sync_copy(movies_hbm.at[pl.ds(base, BPW)], vidx_m)

    for c in range(NUM_FACTORS):
        for b in range(NCH):
            sl = pl.ds(b * CHUNK, CHUNK)
            pltpu.async_copy(uft_hbm.at[c].at[vidx_u.at[sl]],
                             ucols.at[c, sl], sem_u)
            pltpu.async_copy(mft_hbm.at[c].at[vidx_m.at[sl]],
                             mcols.at[c, sl], sem_m)
    for _ in range(NUM_FACTORS):
        pltpu.make_async_copy(uft_hbm.at[0, pl.ds(0, BPW)], ucols.at[0], sem_u).wait()
        pltpu.make_async_copy(mft_hbm.at[0, pl.ds(0, BPW)], mcols.at[0], sem_m).wait()

    def group_body(g, carry):
        j = g * 16
        acc = ucols[0, pl.ds(j, 16)] * mcols[0, pl.ds(j, 16)]
        for c in range(1, NUM_FACTORS):
            acc = acc + ucols[c, pl.ds(j, 16)] * mcols[c, pl.ds(j, 16)]
        outv[pl.ds(j, 16)] = acc
        return carry

    lax.fori_loop(0, BPW // NUM_FACTORS, group_body, 0)
    pltpu.sync_copy(outv, out_hbm.at[pl.ds(base, BPW)])


def kernel(data, user_factors, movie_factors):
    users = data[:, 0]
    movies = data[:, 1]
    uf_t = user_factors.T
    mf_t = movie_factors.T
    mesh = plsc.VectorSubcoreMesh(core_axis_name="c", subcore_axis_name="s",
                                  num_cores=NC, num_subcores=NS)
    f = pl.kernel(
        _sc_body,
        out_type=jax.ShapeDtypeStruct((BATCH,), jnp.float32),
        mesh=mesh,
        scratch_types=[
            pltpu.VMEM((BPW,), jnp.int32),
            pltpu.VMEM((BPW,), jnp.int32),
            pltpu.VMEM((NUM_FACTORS, BPW), jnp.float32),
            pltpu.VMEM((NUM_FACTORS, BPW), jnp.float32),
            pltpu.VMEM((BPW,), jnp.float32),
            pltpu.SemaphoreType.DMA,
            pltpu.SemaphoreType.DMA,
        ],
        compiler_params=pltpu.CompilerParams(use_tc_tiling_on_sc=False),
    )
    return f(users, movies, uf_t, mf_t)


# speedup vs baseline: 3.1671x; 3.1671x over previous
import jax
import jax.numpy as jnp
from jax import lax
from jax.experimental import pallas as pl
from jax.experimental.pallas import tpu as pltpu, tpu_sc as plsc

NUM_FACTORS = 16
BATCH = 16384
ROWS_PER_BLOCK = 8
BLOCK = ROWS_PER_BLOCK * NUM_FACTORS
NBLOCKS = 1000000 // ROWS_PER_BLOCK
NC, NS = 2, 16
NW = NC * NS
BPW = BATCH // NW
CHUNK = 128
NCH = BPW // CHUNK


def _sc_body(users_hbm, movies_hbm, uf_hbm, mf_hbm, out_hbm,
             vidx_u, vidx_m, bix_u, bix_m, ublk, mblk, outv, sem_u, sem_m):
    wid = lax.axis_index("s") * NC + lax.axis_index("c")
    base = wid * BPW
    pltpu.sync_copy(users_hbm.at[pl.ds(base, BPW)], vidx_u)
    pltpu.sync_copy(movies_hbm.at[pl.ds(base, BPW)], vidx_m)

    def build(g, carry):
        j = g * 16
        bix_u[pl.ds(j, 16)] = vidx_u[pl.ds(j, 16)] >> 3
        bix_m[pl.ds(j, 16)] = vidx_m[pl.ds(j, 16)] >> 3
        return carry

    lax.fori_loop(0, BPW // 16, build, 0)

    def fire(c):
        sl = pl.ds(c * CHUNK, CHUNK)
        slot = c % 2
        pltpu.async_copy(uf_hbm.at[bix_u.at[sl]], ublk.at[slot], sem_u)
        pltpu.async_copy(mf_hbm.at[bix_m.at[sl]], mblk.at[slot], sem_m)

    def drain():
        pltpu.make_async_copy(uf_hbm.at[pl.ds(0, CHUNK)], ublk.at[0], sem_u).wait()
        pltpu.make_async_copy(mf_hbm.at[pl.ds(0, CHUNK)], mblk.at[0], sem_m).wait()

    iota16 = lax.broadcasted_iota(jnp.int32, (NUM_FACTORS,), 0)
    dn = lax.GatherDimensionNumbers(
        offset_dims=(), collapsed_slice_dims=(0,), start_index_map=(0,))

    def perm(v, k):
        return lax.gather(v, (iota16 ^ k)[:, None], dn, slice_sizes=(1,),
                          mode=lax.GatherScatterMode.PROMISE_IN_BOUNDS)

    def compute(c):
        slot = c % 2

        def group_body(g, carry):
            j = c * CHUNK + g * 16
            qu = (vidx_u[pl.ds(j, 16)] & 7) * NUM_FACTORS
            qm = (vidx_m[pl.ds(j, 16)] & 7) * NUM_FACTORS
            acc = jnp.zeros((NUM_FACTORS,), jnp.float32)
            for t in range(16):
                u = ublk[slot, g * 16 + t, pl.ds(qu[t], NUM_FACTORS)]
                m = mblk[slot, g * 16 + t, pl.ds(qm[t], NUM_FACTORS)]
                p = u * m
                for k in (1, 2, 4, 8):
                    p = p + perm(p, k)
                acc = jnp.where(iota16 == t, p, acc)
            outv[pl.ds(j, 16)] = acc
            return carry

        lax.fori_loop(0, CHUNK // 16, group_body, 0)

    fire(0)
    for c in range(NCH):
        if c + 1 < NCH:
            fire(c + 1)
        drain()
        compute(c)

    pltpu.sync_copy(outv, out_hbm.at[pl.ds(base, BPW)])


def kernel(data, user_factors, movie_factors):
    users = data[:, 0]
    movies = data[:, 1]
    uf_b = user_factors.reshape(NBLOCKS, BLOCK)
    mf_b = movie_factors.reshape(NBLOCKS, BLOCK)
    mesh = plsc.VectorSubcoreMesh(core_axis_name="c", subcore_axis_name="s",
                                  num_cores=NC, num_subcores=NS)
    f = pl.kernel(
        _sc_body,
        out_type=jax.ShapeDtypeStruct((BATCH,), jnp.float32),
        mesh=mesh,
        scratch_types=[
            pltpu.VMEM((BPW,), jnp.int32),
            pltpu.VMEM((BPW,), jnp.int32),
            pltpu.VMEM((BPW,), jnp.int32),
            pltpu.VMEM((BPW,), jnp.int32),
            pltpu.VMEM((2, CHUNK, BLOCK), jnp.float32),
            pltpu.VMEM((2, CHUNK, BLOCK), jnp.float32),
            pltpu.VMEM((BPW,), jnp.float32),
            pltpu.SemaphoreType.DMA,
            pltpu.SemaphoreType.DMA,
        ],
        compiler_params=pltpu.CompilerParams(use_tc_tiling_on_sc=True),
    )
    return f(users, movies, uf_b, mf_b)
